# Initial kernel scaffold; baseline (speedup 1.0000x reference)
import jax
import jax.numpy as jnp
from jax.experimental import pallas as pl


def kernel(features, adjs, W, b):
    raise NotImplementedError("write your pallas kernel here")


# speedup vs baseline: 1.0556x; 1.0556x over previous
import functools

import jax
import jax.numpy as jnp
from jax.experimental import pallas as pl
from jax.experimental.pallas import tpu as pltpu


def _pick_block(n: int) -> int:
    for bm in range(min(n, 512), 7, -1):
        if n % bm == 0 and bm % 8 == 0:
            return bm
    return n


def _gcn_kernel(x_ref, w_ref, adj_ref, b_ref, out_ref, h_ref):
    s_count = w_ref.shape[0]

    @pl.when(pl.program_id(0) == 0)
    def _compute_h():
        for s in range(s_count):
            h_ref[s] = jnp.dot(
                x_ref[...], w_ref[s], preferred_element_type=jnp.float32
            )

    acc = jnp.dot(adj_ref[0], h_ref[0], preferred_element_type=jnp.float32)
    for s in range(1, s_count):
        acc = acc + jnp.dot(
            adj_ref[s], h_ref[s], preferred_element_type=jnp.float32
        )
    out = acc + b_ref[...]
    norm = jnp.sqrt(jnp.sum(out * out, axis=1, keepdims=True))
    out_ref[...] = out / jnp.maximum(norm, 1e-12)


@functools.partial(jax.jit, static_argnames=())
def kernel(features, adjs, W, b):
    n, d_in = features.shape
    s_count, _, d_out = W.shape
    bm = _pick_block(n)
    grid = (n // bm,)
    b2d = b.reshape(1, d_out)

    return pl.pallas_call(
        _gcn_kernel,
        grid=grid,
        in_specs=[
            pl.BlockSpec((n, d_in), lambda i: (0, 0)),
            pl.BlockSpec((s_count, d_in, d_out), lambda i: (0, 0, 0)),
            pl.BlockSpec((s_count, bm, n), lambda i: (0, i, 0)),
            pl.BlockSpec((1, d_out), lambda i: (0, 0)),
        ],
        out_specs=pl.BlockSpec((bm, d_out), lambda i: (i, 0)),
        out_shape=jax.ShapeDtypeStruct((n, d_out), features.dtype),
        scratch_shapes=[pltpu.VMEM((s_count, n, d_out), jnp.float32)],
        compiler_params=pltpu.CompilerParams(
            dimension_semantics=("arbitrary",),
            vmem_limit_bytes=110 * 1024 * 1024,
        ),
    )(features, W, adjs, b2d)
